# baseline (speedup 1.0000x reference)
import functools

import jax
import jax.numpy as jnp
from jax import lax
from jax.experimental import pallas as pl
from jax.experimental.pallas import tpu as pltpu
from jax.experimental.pallas import tpu_sc as plsc

B_TOTAL = 4096 * 50
NUM_ROWS = 1000
D = 768
NUM_WORKERS = 32
B_PER_W = B_TOTAL // NUM_WORKERS

CHUNK = 40
NBUF = 2
S_ROWS = B_PER_W // 2
N_GROUPS = S_ROWS // (CHUNK * NBUF)

BLK = 16
P_BASE = S_ROWS
ROW_BLOCKS_PER_GROUP = (B_PER_W - S_ROWS) // (BLK * N_GROUPS)
LOOKAHEAD = 10

_mesh = plsc.VectorSubcoreMesh(core_axis_name="c", subcore_axis_name="s")


@functools.partial(
    pl.kernel,
    mesh=_mesh,
    out_type=jax.ShapeDtypeStruct((B_TOTAL, D), jnp.float32),
    scratch_types=(
        [pltpu.VMEM((B_PER_W,), jnp.int32)]
        + [pltpu.VMEM((CHUNK, D), jnp.float32) for _ in range(NBUF)]
        + [pltpu.VMEM_SHARED((NUM_ROWS, D), jnp.float32)]
        + [pltpu.SemaphoreType.DMA for _ in range(2 * NBUF + 1)]
    ),
)
def _gather_sc(ids_hbm, table_hbm, out_hbm, idx_v, *rest):
    bufs = rest[:NBUF]
    table_sp = rest[NBUF]
    gsems = rest[NBUF + 1:2 * NBUF + 1]
    ssems = rest[2 * NBUF + 1:3 * NBUF + 1]
    rsem = rest[3 * NBUF + 1]

    cid = lax.axis_index("c")
    sid = lax.axis_index("s")
    wid = sid * 2 + cid
    base = wid * B_PER_W

    @pl.when(sid == 0)
    def _():
        pltpu.sync_copy(table_hbm, table_sp)

    pltpu.sync_copy(ids_hbm.at[pl.ds(base, B_PER_W)], idx_v)
    plsc.subcore_barrier()

    def gather_copy(chunk_id, b):
        off = chunk_id * CHUNK
        return pltpu.make_async_copy(
            table_hbm.at[idx_v.at[pl.ds(off, CHUNK)]], bufs[b], gsems[b]
        )

    def scatter_copy(chunk_id, b):
        off = chunk_id * CHUNK
        return pltpu.make_async_copy(
            bufs[b], out_hbm.at[pl.ds(base + off, CHUNK)], ssems[b]
        )

    def row_copy(row_id, out_row):
        return pltpu.make_async_copy(
            table_sp.at[row_id], out_hbm.at[out_row], rsem
        )

    for b in range(NBUF):
        gather_copy(b, b).start()

    def group(g, carry):
        for k in range(ROW_BLOCKS_PER_GROUP):
            blk = g * ROW_BLOCKS_PER_GROUP + k
            off = P_BASE + blk * BLK
            idx_vec = idx_v[pl.ds(off, BLK)]
            for j in range(BLK):
                row_copy(idx_vec[j], base + off + j).start()
            @pl.when(blk >= LOOKAHEAD)
            def _():
                for j in range(BLK):
                    row_copy(0, 0).wait()
        c0 = g * NBUF
        for b in range(NBUF):
            gather_copy(c0 + b, b).wait()
            scatter_copy(c0 + b, b).start()
        for b in range(NBUF):
            scatter_copy(c0 + b, b).wait()
            @pl.when(g < N_GROUPS - 1)
            def _():
                gather_copy(c0 + NBUF + b, b).start()
        return carry

    lax.fori_loop(0, N_GROUPS, group, 0)

    for _ in range(LOOKAHEAD):
        for j in range(BLK):
            row_copy(0, 0).wait()


def kernel(input_ids, weight):
    n_batch, n_seq = input_ids.shape
    ids_flat = input_ids.astype(jnp.int32).T.reshape(-1)
    out = _gather_sc(ids_flat, weight)
    return out.reshape(n_seq, n_batch, D).transpose(1, 0, 2)


# speedup vs baseline: 1.0587x; 1.0587x over previous
import functools

import jax
import jax.numpy as jnp
from jax import lax
from jax.experimental import pallas as pl
from jax.experimental.pallas import tpu as pltpu
from jax.experimental.pallas import tpu_sc as plsc

B_TOTAL = 4096 * 50
NUM_ROWS = 1000
D = 768
NUM_WORKERS = 32
B_PER_W = B_TOTAL // NUM_WORKERS

CHUNK = 48
NBUF = 2
S_ROWS = 2400
N_GROUPS = S_ROWS // (CHUNK * NBUF)

BLK = 16
P_BASE = S_ROWS
ROW_BLOCKS_PER_GROUP = (B_PER_W - S_ROWS) // (BLK * N_GROUPS)
LOOKAHEAD = 10

_mesh = plsc.VectorSubcoreMesh(core_axis_name="c", subcore_axis_name="s")


@functools.partial(
    pl.kernel,
    mesh=_mesh,
    out_type=jax.ShapeDtypeStruct((B_TOTAL, D), jnp.float32),
    scratch_types=(
        [pltpu.VMEM((B_PER_W,), jnp.int32)]
        + [pltpu.VMEM((CHUNK, D), jnp.float32) for _ in range(NBUF)]
        + [pltpu.VMEM_SHARED((NUM_ROWS, D), jnp.float32)]
        + [pltpu.SemaphoreType.DMA for _ in range(2 * NBUF + 1)]
    ),
)
def _gather_sc(ids_hbm, table_hbm, out_hbm, idx_v, *rest):
    bufs = rest[:NBUF]
    table_sp = rest[NBUF]
    gsems = rest[NBUF + 1:2 * NBUF + 1]
    ssems = rest[2 * NBUF + 1:3 * NBUF + 1]
    rsem = rest[3 * NBUF + 1]

    cid = lax.axis_index("c")
    sid = lax.axis_index("s")
    wid = sid * 2 + cid
    base = wid * B_PER_W

    @pl.when(sid == 0)
    def _():
        pltpu.sync_copy(table_hbm, table_sp)

    pltpu.sync_copy(ids_hbm.at[pl.ds(base, B_PER_W)], idx_v)
    plsc.subcore_barrier()

    def gather_copy(chunk_id, b):
        off = chunk_id * CHUNK
        return pltpu.make_async_copy(
            table_hbm.at[idx_v.at[pl.ds(off, CHUNK)]], bufs[b], gsems[b]
        )

    def scatter_copy(chunk_id, b):
        off = chunk_id * CHUNK
        return pltpu.make_async_copy(
            bufs[b], out_hbm.at[pl.ds(base + off, CHUNK)], ssems[b]
        )

    def row_copy(row_id, out_row):
        return pltpu.make_async_copy(
            table_sp.at[row_id], out_hbm.at[out_row], rsem
        )

    for b in range(NBUF):
        gather_copy(b, b).start()

    def group(g, carry):
        for k in range(ROW_BLOCKS_PER_GROUP):
            blk = g * ROW_BLOCKS_PER_GROUP + k
            off = P_BASE + blk * BLK
            idx_vec = idx_v[pl.ds(off, BLK)]
            for j in range(BLK):
                row_copy(idx_vec[j], base + off + j).start()
            @pl.when(blk >= LOOKAHEAD)
            def _():
                for j in range(BLK):
                    row_copy(0, 0).wait()
        c0 = g * NBUF
        for b in range(NBUF):
            gather_copy(c0 + b, b).wait()
            scatter_copy(c0 + b, b).start()
        for b in range(NBUF):
            scatter_copy(c0 + b, b).wait()
            @pl.when(g < N_GROUPS - 1)
            def _():
                gather_copy(c0 + NBUF + b, b).start()
        return carry

    lax.fori_loop(0, N_GROUPS, group, 0)

    for _ in range(LOOKAHEAD):
        for j in range(BLK):
            row_copy(0, 0).wait()


def kernel(input_ids, weight):
    n_batch, n_seq = input_ids.shape
    ids_flat = input_ids.astype(jnp.int32).T.reshape(-1)
    out = _gather_sc(ids_flat, weight)
    return out.reshape(n_seq, n_batch, D).transpose(1, 0, 2)
